# baseline (speedup 1.0000x reference)
import functools

import jax
import jax.numpy as jnp
from jax import lax
from jax.experimental import pallas as pl
from jax.experimental.pallas import tpu as pltpu
from jax.experimental.pallas import tpu_sc as plsc

_EPS = 1e-12


_NUM_CORES = 2
_NUM_SUBCORES = 16
_NUM_WORKERS = _NUM_CORES * _NUM_SUBCORES


def _sc_gather(table, ids_flat, chunk, piece0, npiece):
    h = table.shape[1]
    tok_per_w = npiece // _NUM_WORKERS
    n_chunks = tok_per_w // chunk
    mesh = plsc.VectorSubcoreMesh(core_axis_name="c", subcore_axis_name="s")

    @functools.partial(
        pl.kernel,
        mesh=mesh,
        out_type=jax.ShapeDtypeStruct((npiece, h), jnp.float32),
        scratch_types=[
            pltpu.VMEM((tok_per_w,), jnp.int32),
            pltpu.VMEM((chunk, h), jnp.float32),
            pltpu.VMEM((chunk, h), jnp.float32),
            pltpu.SemaphoreType.DMA,
            pltpu.SemaphoreType.DMA,
        ],
    )
    def gather_kernel(table_hbm, idx_hbm, out_hbm, idx_v, buf0, buf1, sem0, sem1):
        wid = lax.axis_index("s") * _NUM_CORES + lax.axis_index("c")
        base = wid * tok_per_w
        pltpu.sync_copy(idx_hbm.at[pl.ds(piece0 + base, tok_per_w)], idx_v)

        bufs = (buf0, buf1)
        sems = (sem0, sem1)

        pltpu.async_copy(table_hbm.at[idx_v.at[pl.ds(0, chunk)]], buf0, sem0)

        def body(i, _):
            for p in range(2):
                nxt = i + 1

                @pl.when(jnp.logical_and(nxt % 2 == p, nxt < n_chunks))
                def _():
                    pltpu.async_copy(
                        table_hbm.at[idx_v.at[pl.ds(nxt * chunk, chunk)]],
                        bufs[p],
                        sems[p],
                    )

            for p in range(2):

                @pl.when(i % 2 == p)
                def _():
                    pltpu.make_async_copy(
                        table_hbm.at[idx_v.at[pl.ds(i * chunk, chunk)]],
                        bufs[p],
                        sems[p],
                    ).wait()
                    pltpu.sync_copy(
                        bufs[p], out_hbm.at[pl.ds(base + i * chunk, chunk)]
                    )

            return 0

        lax.fori_loop(0, n_chunks, body, 0)

    return gather_kernel(table, ids_flat)


def _ln_body(carry_ref, w_ref, tt_ref, pos_ref, type_ref, lnw_ref, lnb_ref,
             o_ref):
    del carry_ref
    x = w_ref[0]
    tt = tt_ref[0, 0, :].astype(jnp.float32)
    t0 = type_ref[0, :]
    dt = type_ref[1, :] - t0
    x = x + pos_ref[...] + t0[None, :] + tt[:, None] * dt[None, :]
    u = jnp.mean(x, axis=-1, keepdims=True)
    xc = x - u
    v = jnp.mean(xc * xc, axis=-1, keepdims=True)
    y = xc * lax.rsqrt(v + _EPS)
    o_ref[0] = y * lnw_ref[...][None, :] + lnb_ref[...][None, :]


def _ln_body_first(w_ref, tt_ref, pos_ref, type_ref, lnw_ref, lnb_ref, o_ref):
    _ln_body(None, w_ref, tt_ref, pos_ref, type_ref, lnw_ref, lnb_ref, o_ref)


def _tc_layernorm_piece(carry, bt, w_rows, tt3, pos_table, type_table, ln_w,
                        ln_b, b0):
    bp, s, h = w_rows.shape
    specs = [
        pl.BlockSpec((1, s, h), lambda i: (i, 0, 0)),
        pl.BlockSpec((1, 1, s), lambda i, b0=b0: (b0 + i, 0, 0)),
        pl.BlockSpec((s, h), lambda i: (0, 0)),
        pl.BlockSpec((2, h), lambda i: (0, 0)),
        pl.BlockSpec((h,), lambda i: (0,)),
        pl.BlockSpec((h,), lambda i: (0,)),
    ]
    common = dict(
        grid=(bp,),
        out_specs=pl.BlockSpec((1, s, h), lambda i, b0=b0: (b0 + i, 0, 0)),
        out_shape=jax.ShapeDtypeStruct((bt, s, h), jnp.float32),
    )
    if carry is None:
        return pl.pallas_call(_ln_body_first, in_specs=specs, **common)(
            w_rows, tt3, pos_table, type_table, ln_w, ln_b)
    return pl.pallas_call(
        _ln_body,
        in_specs=[pl.BlockSpec(memory_space=pl.ANY)] + specs,
        input_output_aliases={0: 0},
        **common,
    )(carry, w_rows, tt3, pos_table, type_table, ln_w, ln_b)


_PIECE_SIZES = (8, 12, 8, 4)


def kernel(input_ids, token_type_ids, word_table, pos_table, type_table,
           ln_weight, ln_bias):
    b, s = input_ids.shape
    h = word_table.shape[1]
    ids_flat = input_ids.reshape(-1).astype(jnp.int32)
    tt3 = token_type_ids.reshape(b, 1, s).astype(jnp.int32)

    offs = [sum(_PIECE_SIZES[:p]) for p in range(len(_PIECE_SIZES))]
    rows = [
        _sc_gather(word_table, ids_flat, chunk=64, piece0=b0 * s,
                   npiece=bp * s)
        for b0, bp in zip(offs, _PIECE_SIZES)
    ]
    out = None
    for r, b0, bp in zip(rows, offs, _PIECE_SIZES):
        out = _tc_layernorm_piece(
            out, b, r.reshape(bp, s, h), tt3,
            pos_table, type_table, ln_weight, ln_bias, b0)
    return out


# speedup vs baseline: 1.0093x; 1.0093x over previous
import functools

import jax
import jax.numpy as jnp
from jax import lax
from jax.experimental import pallas as pl
from jax.experimental.pallas import tpu as pltpu
from jax.experimental.pallas import tpu_sc as plsc

_EPS = 1e-12


_NUM_CORES = 2
_NUM_SUBCORES = 16
_NUM_WORKERS = _NUM_CORES * _NUM_SUBCORES


def _sc_gather(table, ids_flat, chunk, piece0, npiece):
    h = table.shape[1]
    tok_per_w = npiece // _NUM_WORKERS
    n_chunks = tok_per_w // chunk
    mesh = plsc.VectorSubcoreMesh(core_axis_name="c", subcore_axis_name="s")

    @functools.partial(
        pl.kernel,
        mesh=mesh,
        out_type=jax.ShapeDtypeStruct((npiece, h), jnp.float32),
        scratch_types=[
            pltpu.VMEM((tok_per_w,), jnp.int32),
            pltpu.VMEM((chunk, h), jnp.float32),
            pltpu.VMEM((chunk, h), jnp.float32),
            pltpu.SemaphoreType.DMA,
            pltpu.SemaphoreType.DMA,
        ],
    )
    def gather_kernel(table_hbm, idx_hbm, out_hbm, idx_v, buf0, buf1, sem0, sem1):
        wid = lax.axis_index("s") * _NUM_CORES + lax.axis_index("c")
        base = wid * tok_per_w
        pltpu.sync_copy(idx_hbm.at[pl.ds(piece0 + base, tok_per_w)], idx_v)

        bufs = (buf0, buf1)
        sems = (sem0, sem1)

        pltpu.async_copy(table_hbm.at[idx_v.at[pl.ds(0, chunk)]], buf0, sem0)

        def body(i, _):
            for p in range(2):
                nxt = i + 1

                @pl.when(jnp.logical_and(nxt % 2 == p, nxt < n_chunks))
                def _():
                    pltpu.async_copy(
                        table_hbm.at[idx_v.at[pl.ds(nxt * chunk, chunk)]],
                        bufs[p],
                        sems[p],
                    )

            for p in range(2):

                @pl.when(i % 2 == p)
                def _():
                    pltpu.make_async_copy(
                        table_hbm.at[idx_v.at[pl.ds(i * chunk, chunk)]],
                        bufs[p],
                        sems[p],
                    ).wait()
                    pltpu.sync_copy(
                        bufs[p], out_hbm.at[pl.ds(base + i * chunk, chunk)]
                    )

            return 0

        lax.fori_loop(0, n_chunks, body, 0)

    return gather_kernel(table, ids_flat)


def _ln_body(carry_ref, w_ref, tt_ref, pos_ref, type_ref, lnw_ref, lnb_ref,
             o_ref):
    del carry_ref
    x = w_ref[0]
    tt = tt_ref[0, 0, :].astype(jnp.float32)
    t0 = type_ref[0, :]
    dt = type_ref[1, :] - t0
    x = x + pos_ref[...] + t0[None, :] + tt[:, None] * dt[None, :]
    u = jnp.mean(x, axis=-1, keepdims=True)
    xc = x - u
    v = jnp.mean(xc * xc, axis=-1, keepdims=True)
    y = xc * lax.rsqrt(v + _EPS)
    o_ref[0] = y * lnw_ref[...][None, :] + lnb_ref[...][None, :]


def _ln_body_first(w_ref, tt_ref, pos_ref, type_ref, lnw_ref, lnb_ref, o_ref):
    _ln_body(None, w_ref, tt_ref, pos_ref, type_ref, lnw_ref, lnb_ref, o_ref)


def _tc_layernorm_piece(carry, bt, w_rows, tt3, pos_table, type_table, ln_w,
                        ln_b, b0):
    bp, s, h = w_rows.shape
    specs = [
        pl.BlockSpec((1, s, h), lambda i: (i, 0, 0)),
        pl.BlockSpec((1, 1, s), lambda i, b0=b0: (b0 + i, 0, 0)),
        pl.BlockSpec((s, h), lambda i: (0, 0)),
        pl.BlockSpec((2, h), lambda i: (0, 0)),
        pl.BlockSpec((h,), lambda i: (0,)),
        pl.BlockSpec((h,), lambda i: (0,)),
    ]
    common = dict(
        grid=(bp,),
        out_specs=pl.BlockSpec((1, s, h), lambda i, b0=b0: (b0 + i, 0, 0)),
        out_shape=jax.ShapeDtypeStruct((bt, s, h), jnp.float32),
    )
    if carry is None:
        return pl.pallas_call(_ln_body_first, in_specs=specs, **common)(
            w_rows, tt3, pos_table, type_table, ln_w, ln_b)
    return pl.pallas_call(
        _ln_body,
        in_specs=[pl.BlockSpec(memory_space=pl.ANY)] + specs,
        input_output_aliases={0: 0},
        **common,
    )(carry, w_rows, tt3, pos_table, type_table, ln_w, ln_b)


_PIECE_SIZES = (8, 8, 8, 8)


def kernel(input_ids, token_type_ids, word_table, pos_table, type_table,
           ln_weight, ln_bias):
    b, s = input_ids.shape
    h = word_table.shape[1]
    ids_flat = input_ids.reshape(-1).astype(jnp.int32)
    tt3 = token_type_ids.reshape(b, 1, s).astype(jnp.int32)

    offs = [sum(_PIECE_SIZES[:p]) for p in range(len(_PIECE_SIZES))]
    rows = [
        _sc_gather(word_table, ids_flat, chunk=64, piece0=b0 * s,
                   npiece=bp * s)
        for b0, bp in zip(offs, _PIECE_SIZES)
    ]
    out = None
    for r, b0, bp in zip(rows, offs, _PIECE_SIZES):
        out = _tc_layernorm_piece(
            out, b, r.reshape(bp, s, h), tt3,
            pos_table, type_table, ln_weight, ln_bias, b0)
    return out
